# baseline (speedup 1.0000x reference)
import jax
import jax.numpy as jnp
from jax.experimental import pallas as pl
from jax.experimental.pallas import tpu as pltpu

B = 16
L = 2048
K = 512
CHUNK = 256
TINY = 1.1754943508222875e-38


def _threefry_bits(x1):
    ks = (jnp.uint32(0), jnp.uint32(1), jnp.uint32(0x1BD11BDA) ^ jnp.uint32(1))
    rot = ((13, 15, 26, 6), (17, 29, 16, 24))
    x0 = x1
    x1 = (x1 << jnp.uint32(13)) | (x1 >> jnp.uint32(32 - 13))
    x1 = x0 ^ x1
    first = True
    for i in range(5):
        for r in rot[i % 2]:
            if first:
                first = False
                continue
            x0 = x0 + x1
            x1 = (x1 << jnp.uint32(r)) | (x1 >> jnp.uint32(32 - r))
            x1 = x0 ^ x1
        x0 = x0 + ks[(i + 1) % 3]
        x1 = x1 + (ks[(i + 2) % 3] + jnp.uint32(i + 1))
    return x0 ^ x1


def _body(t_ref, x0_ref, q_ref, out_ref):
    b = pl.program_id(0)
    c = pl.program_id(1)
    x0 = x0_ref[0, 0, :]
    q = q_ref[0]

    col = jax.lax.broadcasted_iota(jnp.int32, (CHUNK, K), 1)
    row = jax.lax.broadcasted_iota(jnp.int32, (CHUNK, K), 0)

    onehot = (x0[:, None] == col).astype(jnp.bfloat16)
    probs = jnp.dot(onehot, q.astype(jnp.bfloat16),
                    preferred_element_type=jnp.float32)
    logits = jnp.log(probs)

    idx = (b * L + c * CHUNK + row) * K + col + 1
    bits = _threefry_bits(idx.astype(jnp.uint32))
    fb = (bits >> jnp.uint32(9)) | jnp.uint32(0x3F800000)
    u = jax.lax.bitcast_convert_type(fb, jnp.float32) - jnp.float32(1.0)
    u = u + jnp.float32(TINY)
    g = -jnp.log(-jnp.log(u))

    scores = logits + g
    m = jnp.max(scores, axis=1, keepdims=True)
    first = jnp.min(jnp.where(scores == m, col, K), axis=1)
    out_ref[0, 0, :] = first.astype(jnp.int32)


def kernel(x_0, t, Qt_bar):
    x0r = x_0.astype(jnp.int32).reshape(B, 1, L)
    tt = t.astype(jnp.int32)
    grid = (B, L // CHUNK)
    out = pl.pallas_call(
        _body,
        grid_spec=pltpu.PrefetchScalarGridSpec(
            num_scalar_prefetch=1,
            grid=grid,
            in_specs=[
                pl.BlockSpec((1, 1, CHUNK), lambda b, c, t_ref: (b, 0, c)),
                pl.BlockSpec((1, K, K), lambda b, c, t_ref: (t_ref[b], 0, 0)),
            ],
            out_specs=pl.BlockSpec((1, 1, CHUNK), lambda b, c, t_ref: (b, 0, c)),
        ),
        out_shape=jax.ShapeDtypeStruct((B, 1, L), jnp.int32),
        compiler_params=pltpu.CompilerParams(
            dimension_semantics=("parallel", "parallel")),
    )(tt, x0r, Qt_bar)
    return out.reshape(B, L)


# speedup vs baseline: 1.0623x; 1.0623x over previous
import jax
import jax.numpy as jnp
from jax.experimental import pallas as pl
from jax.experimental.pallas import tpu as pltpu

B = 16
L = 2048
K = 512
CHUNK = 512
TINY = 1.1754943508222875e-38


def _threefry_bits(x1):
    ks = (jnp.uint32(0), jnp.uint32(1), jnp.uint32(0x1BD11BDA) ^ jnp.uint32(1))
    rot = ((13, 15, 26, 6), (17, 29, 16, 24))
    x0 = x1
    x1 = (x1 << jnp.uint32(13)) | (x1 >> jnp.uint32(32 - 13))
    x1 = x0 ^ x1
    first = True
    for i in range(5):
        for r in rot[i % 2]:
            if first:
                first = False
                continue
            x0 = x0 + x1
            x1 = (x1 << jnp.uint32(r)) | (x1 >> jnp.uint32(32 - r))
            x1 = x0 ^ x1
        x0 = x0 + ks[(i + 1) % 3]
        x1 = x1 + (ks[(i + 2) % 3] + jnp.uint32(i + 1))
    return x0 ^ x1


SUB = 4


def _body(t_ref, x0_ref, q_ref, out_ref):
    b = pl.program_id(0)
    c = pl.program_id(1)
    q16 = q_ref[0].astype(jnp.bfloat16)

    rows = CHUNK // SUB
    col = jax.lax.broadcasted_iota(jnp.int32, (rows, K), 1)
    row = jax.lax.broadcasted_iota(jnp.int32, (rows, K), 0)
    for s in range(SUB):
        x0 = x0_ref[0, 0, pl.ds(s * rows, rows)]

        onehot = (x0[:, None] == col).astype(jnp.bfloat16)
        probs = jnp.dot(onehot, q16, preferred_element_type=jnp.float32)
        logits = jnp.log(probs)

        idx = (b * L + c * CHUNK + s * rows + row) * K + col + 1
        bits = _threefry_bits(idx.astype(jnp.uint32))
        fb = (bits >> jnp.uint32(9)) | jnp.uint32(0x3F800000)
        u = jax.lax.bitcast_convert_type(fb, jnp.float32) - jnp.float32(1.0)
        u = u + jnp.float32(TINY)
        g = -jnp.log(-jnp.log(u))

        scores = logits + g
        m = jnp.max(scores, axis=1, keepdims=True)
        first = jnp.min(jnp.where(scores == m, col, K), axis=1)
        out_ref[0, 0, pl.ds(s * rows, rows)] = first.astype(jnp.int32)


def kernel(x_0, t, Qt_bar):
    x0r = x_0.astype(jnp.int32).reshape(B, 1, L)
    tt = t.astype(jnp.int32)
    grid = (B, L // CHUNK)
    out = pl.pallas_call(
        _body,
        grid_spec=pltpu.PrefetchScalarGridSpec(
            num_scalar_prefetch=1,
            grid=grid,
            in_specs=[
                pl.BlockSpec((1, 1, CHUNK), lambda b, c, t_ref: (b, 0, c)),
                pl.BlockSpec((1, K, K), lambda b, c, t_ref: (t_ref[b], 0, 0)),
            ],
            out_specs=pl.BlockSpec((1, 1, CHUNK), lambda b, c, t_ref: (b, 0, c)),
        ),
        out_shape=jax.ShapeDtypeStruct((B, 1, L), jnp.int32),
        compiler_params=pltpu.CompilerParams(
            dimension_semantics=("parallel", "parallel")),
    )(tt, x0r, Qt_bar)
    return out.reshape(B, L)


# speedup vs baseline: 1.0858x; 1.0221x over previous
import jax
import jax.numpy as jnp
from jax.experimental import pallas as pl
from jax.experimental.pallas import tpu as pltpu

B = 16
L = 2048
K = 512
CHUNK = 1024
TINY = 1.1754943508222875e-38


def _threefry_bits(x1):
    ks = (jnp.uint32(0), jnp.uint32(1), jnp.uint32(0x1BD11BDA) ^ jnp.uint32(1))
    rot = ((13, 15, 26, 6), (17, 29, 16, 24))
    x0 = x1
    x1 = (x1 << jnp.uint32(13)) | (x1 >> jnp.uint32(32 - 13))
    x1 = x0 ^ x1
    first = True
    for i in range(5):
        for r in rot[i % 2]:
            if first:
                first = False
                continue
            x0 = x0 + x1
            x1 = (x1 << jnp.uint32(r)) | (x1 >> jnp.uint32(32 - r))
            x1 = x0 ^ x1
        x0 = x0 + ks[(i + 1) % 3]
        x1 = x1 + (ks[(i + 2) % 3] + jnp.uint32(i + 1))
    return x0 ^ x1


SUB = 8


def _body(t_ref, x0_ref, q_ref, out_ref):
    b = pl.program_id(0)
    c = pl.program_id(1)
    q16 = q_ref[0].astype(jnp.bfloat16)

    rows = CHUNK // SUB
    col = jax.lax.broadcasted_iota(jnp.int32, (rows, K), 1)
    row = jax.lax.broadcasted_iota(jnp.int32, (rows, K), 0)
    for s in range(SUB):
        x0 = x0_ref[0, 0, pl.ds(s * rows, rows)]

        onehot = (x0[:, None] == col).astype(jnp.bfloat16)
        probs = jnp.dot(onehot, q16, preferred_element_type=jnp.float32)
        logits = jnp.log(probs)

        idx = (b * L + c * CHUNK + s * rows + row) * K + col + 1
        bits = _threefry_bits(idx.astype(jnp.uint32))
        fb = (bits >> jnp.uint32(9)) | jnp.uint32(0x3F800000)
        u = jax.lax.bitcast_convert_type(fb, jnp.float32) - jnp.float32(1.0)
        u = u + jnp.float32(TINY)
        g = -jnp.log(-jnp.log(u))

        scores = logits + g
        m = jnp.max(scores, axis=1, keepdims=True)
        first = jnp.min(jnp.where(scores == m, col, K), axis=1)
        out_ref[0, 0, pl.ds(s * rows, rows)] = first.astype(jnp.int32)


def kernel(x_0, t, Qt_bar):
    x0r = x_0.astype(jnp.int32).reshape(B, 1, L)
    tt = t.astype(jnp.int32)
    grid = (B, L // CHUNK)
    out = pl.pallas_call(
        _body,
        grid_spec=pltpu.PrefetchScalarGridSpec(
            num_scalar_prefetch=1,
            grid=grid,
            in_specs=[
                pl.BlockSpec((1, 1, CHUNK), lambda b, c, t_ref: (b, 0, c)),
                pl.BlockSpec((1, K, K), lambda b, c, t_ref: (t_ref[b], 0, 0)),
            ],
            out_specs=pl.BlockSpec((1, 1, CHUNK), lambda b, c, t_ref: (b, 0, c)),
        ),
        out_shape=jax.ShapeDtypeStruct((B, 1, L), jnp.int32),
        compiler_params=pltpu.CompilerParams(
            dimension_semantics=("parallel", "parallel")),
    )(tt, x0r, Qt_bar)
    return out.reshape(B, L)


# speedup vs baseline: 1.0956x; 1.0090x over previous
import jax
import jax.numpy as jnp
from jax.experimental import pallas as pl
from jax.experimental.pallas import tpu as pltpu

B = 16
L = 2048
K = 512
CHUNK = 2048
TINY = 1.1754943508222875e-38


def _threefry_bits(x1):
    ks = (jnp.uint32(0), jnp.uint32(1), jnp.uint32(0x1BD11BDA) ^ jnp.uint32(1))
    rot = ((13, 15, 26, 6), (17, 29, 16, 24))
    x0 = x1
    x1 = (x1 << jnp.uint32(13)) | (x1 >> jnp.uint32(32 - 13))
    x1 = x0 ^ x1
    first = True
    for i in range(5):
        for r in rot[i % 2]:
            if first:
                first = False
                continue
            x0 = x0 + x1
            x1 = (x1 << jnp.uint32(r)) | (x1 >> jnp.uint32(32 - r))
            x1 = x0 ^ x1
        x0 = x0 + ks[(i + 1) % 3]
        x1 = x1 + (ks[(i + 2) % 3] + jnp.uint32(i + 1))
    return x0 ^ x1


SUB = 16


def _body(t_ref, x0_ref, q_ref, out_ref):
    b = pl.program_id(0)
    c = pl.program_id(1)
    q16 = q_ref[0].astype(jnp.bfloat16)

    rows = CHUNK // SUB
    col = jax.lax.broadcasted_iota(jnp.int32, (rows, K), 1)
    row = jax.lax.broadcasted_iota(jnp.int32, (rows, K), 0)
    for s in range(SUB):
        x0 = x0_ref[0, 0, pl.ds(s * rows, rows)]

        onehot = (x0[:, None] == col).astype(jnp.bfloat16)
        probs = jnp.dot(onehot, q16, preferred_element_type=jnp.float32)
        logits = jnp.log(probs)

        idx = (b * L + c * CHUNK + s * rows + row) * K + col + 1
        bits = _threefry_bits(idx.astype(jnp.uint32))
        fb = (bits >> jnp.uint32(9)) | jnp.uint32(0x3F800000)
        u = jax.lax.bitcast_convert_type(fb, jnp.float32) - jnp.float32(1.0)
        u = u + jnp.float32(TINY)
        g = -jnp.log(-jnp.log(u))

        scores = logits + g
        m = jnp.max(scores, axis=1, keepdims=True)
        first = jnp.min(jnp.where(scores == m, col, K), axis=1)
        out_ref[0, 0, pl.ds(s * rows, rows)] = first.astype(jnp.int32)


def kernel(x_0, t, Qt_bar):
    x0r = x_0.astype(jnp.int32).reshape(B, 1, L)
    tt = t.astype(jnp.int32)
    grid = (B, L // CHUNK)
    out = pl.pallas_call(
        _body,
        grid_spec=pltpu.PrefetchScalarGridSpec(
            num_scalar_prefetch=1,
            grid=grid,
            in_specs=[
                pl.BlockSpec((1, 1, CHUNK), lambda b, c, t_ref: (b, 0, c)),
                pl.BlockSpec((1, K, K), lambda b, c, t_ref: (t_ref[b], 0, 0)),
            ],
            out_specs=pl.BlockSpec((1, 1, CHUNK), lambda b, c, t_ref: (b, 0, c)),
        ),
        out_shape=jax.ShapeDtypeStruct((B, 1, L), jnp.int32),
        compiler_params=pltpu.CompilerParams(
            dimension_semantics=("parallel", "parallel")),
    )(tt, x0r, Qt_bar)
    return out.reshape(B, L)
